# Initial kernel scaffold; baseline (speedup 1.0000x reference)
import jax
import jax.numpy as jnp
from jax.experimental import pallas as pl


def kernel(inputs):
    raise NotImplementedError("write your pallas kernel here")


# speedup vs baseline: 1.9640x; 1.9640x over previous
import jax
import jax.numpy as jnp
from jax.experimental import pallas as pl

N = 4096
BLK_R = 1024
BLK_C = 128


def _flip_block(x_ref, p_ref, o_ref):
    o_ref[...] = jax.lax.dot(
        x_ref[...], p_ref[...], preferred_element_type=jnp.float32
    )


def kernel(inputs):
    nr = N // BLK_R
    nc = N // BLK_C
    rev = jnp.equal(
        jnp.arange(BLK_C)[:, None] + jnp.arange(BLK_C)[None, :], BLK_C - 1
    ).astype(jnp.float32)
    return pl.pallas_call(
        _flip_block,
        grid=(nr, nc),
        in_specs=[
            pl.BlockSpec((BLK_R, BLK_C), lambda i, j: (i, nc - 1 - j)),
            pl.BlockSpec((BLK_C, BLK_C), lambda i, j: (0, 0)),
        ],
        out_specs=pl.BlockSpec((BLK_R, BLK_C), lambda i, j: (i, j)),
        out_shape=jax.ShapeDtypeStruct((N, N), jnp.float32),
    )(inputs, rev)


# speedup vs baseline: 3.9774x; 2.0252x over previous
import jax
import jax.numpy as jnp
from jax.experimental import pallas as pl

N = 4096
BLK_R = 4096
BLK_C = 128


def _flip_block(x_ref, p_ref, o_ref):
    o_ref[...] = jax.lax.dot(
        x_ref[...], p_ref[...], preferred_element_type=jnp.float32
    )


def kernel(inputs):
    nr = N // BLK_R
    nc = N // BLK_C
    rev = jnp.equal(
        jnp.arange(BLK_C)[:, None] + jnp.arange(BLK_C)[None, :], BLK_C - 1
    ).astype(jnp.float32)
    return pl.pallas_call(
        _flip_block,
        grid=(nr, nc),
        in_specs=[
            pl.BlockSpec((BLK_R, BLK_C), lambda i, j: (i, nc - 1 - j)),
            pl.BlockSpec((BLK_C, BLK_C), lambda i, j: (0, 0)),
        ],
        out_specs=pl.BlockSpec((BLK_R, BLK_C), lambda i, j: (i, j)),
        out_shape=jax.ShapeDtypeStruct((N, N), jnp.float32),
    )(inputs, rev)


# speedup vs baseline: 4.5608x; 1.1467x over previous
import jax
import jax.numpy as jnp
from jax.experimental import pallas as pl

N = 4096
BLK_R = 4096
BLK_C = 256


def _flip_block(x_ref, p_ref, o_ref):
    o_ref[...] = jax.lax.dot(
        x_ref[...], p_ref[...], preferred_element_type=jnp.float32
    )


def kernel(inputs):
    nr = N // BLK_R
    nc = N // BLK_C
    rev = jnp.equal(
        jnp.arange(BLK_C)[:, None] + jnp.arange(BLK_C)[None, :], BLK_C - 1
    ).astype(jnp.float32)
    return pl.pallas_call(
        _flip_block,
        grid=(nr, nc),
        in_specs=[
            pl.BlockSpec((BLK_R, BLK_C), lambda i, j: (i, nc - 1 - j)),
            pl.BlockSpec((BLK_C, BLK_C), lambda i, j: (0, 0)),
        ],
        out_specs=pl.BlockSpec((BLK_R, BLK_C), lambda i, j: (i, j)),
        out_shape=jax.ShapeDtypeStruct((N, N), jnp.float32),
    )(inputs, rev)
